# baseline (speedup 1.0000x reference)
import functools

import jax
import jax.numpy as jnp
from jax import lax
from jax.experimental import pallas as pl
from jax.experimental.pallas import tpu as pltpu
from jax.experimental.pallas import tpu_sc as plsc

N = 8192
D = 256
K = 8192
BN = 512


CK = 8192


def _argmin_body(x_ref, w_ref, idx_ref, wb_ref, wsq_ref):
    @pl.when(pl.program_id(0) == 0)
    def _():
        w = w_ref[...]
        wb_ref[...] = w.astype(jnp.bfloat16)
        wsq_ref[...] = jnp.sum(w * w, axis=1)[None, :]

    x = x_ref[...]
    xb = (x.astype(jnp.bfloat16)) * jnp.bfloat16(-2.0)
    xsq = jnp.sum(x * x, axis=1, keepdims=True)
    s_parts = [
        lax.dot_general(
            xb, wb_ref[pl.ds(c * CK, CK), :], (((1,), (1,)), ((), ())),
            preferred_element_type=jnp.float32,
        )
        for c in range(K // CK)
    ]
    RB = 64
    NSL = K // 128
    lane = lax.broadcasted_iota(jnp.int32, (RB, 128), 1).astype(jnp.float32)
    outs = []
    for r in range(BN // RB):
        xsq_r = xsq[r * RB:(r + 1) * RB, :]
        run_v = run_c = None
        for v in range(NSL):
            c, u = divmod(v, CK // 128)
            sv = s_parts[c][r * RB:(r + 1) * RB, u * 128:(u + 1) * 128]
            wsq_v = wsq_ref[:, pl.ds(v * 128, 128)]
            dv = (xsq_r + sv) + wsq_v
            if run_v is None:
                run_v = dv
                run_c = jnp.zeros((RB, 128), jnp.float32)
            else:
                better = dv < run_v
                run_v = jnp.where(better, dv, run_v)
                run_c = jnp.where(better, jnp.float32(v), run_c)
        dmin = jnp.min(run_v, axis=1, keepdims=True)
        cand = jnp.where(run_v == dmin, run_c * 128.0 + lane, jnp.float32(K))
        outs.append(jnp.min(cand, axis=1))
    idx_ref[...] = jnp.concatenate(outs, axis=0).astype(jnp.int32)


def _nearest_indices(x_flat, W):
    return pl.pallas_call(
        _argmin_body,
        grid=(N // BN,),
        in_specs=[
            pl.BlockSpec((BN, D), lambda i: (i, 0)),
            pl.BlockSpec((K, D), lambda i: (0, 0)),
        ],
        out_specs=pl.BlockSpec((BN,), lambda i: (i,)),
        out_shape=jax.ShapeDtypeStruct((N,), jnp.int32),
        scratch_shapes=[
            pltpu.VMEM((K, D), jnp.bfloat16),
            pltpu.VMEM((1, K), jnp.float32),
        ],
        compiler_params=pltpu.CompilerParams(
            dimension_semantics=("arbitrary",),
        ),
    )(x_flat, W)


def _gather_rows(W, idx):
    info = plsc.get_sparse_core_info()
    nw = info.num_cores * info.num_subcores
    bpw = N // nw
    mesh = plsc.VectorSubcoreMesh(core_axis_name="c", subcore_axis_name="s")

    @functools.partial(
        pl.kernel,
        mesh=mesh,
        out_type=jax.ShapeDtypeStruct((N, D), jnp.float32),
        scratch_types=[
            pltpu.VMEM((bpw,), jnp.int32),
            pltpu.VMEM((bpw, D), jnp.float32),
            pltpu.SemaphoreType.DMA,
        ],
    )
    def k(w_hbm, idx_hbm, out_hbm, idx_v, rows_v, sem):
        wid = lax.axis_index("s") * info.num_cores + lax.axis_index("c")
        base = wid * bpw
        pltpu.sync_copy(idx_hbm.at[pl.ds(base, bpw)], idx_v)
        pltpu.async_copy(w_hbm.at[idx_v], rows_v, sem).wait()
        pltpu.sync_copy(rows_v, out_hbm.at[pl.ds(base, bpw)])

    return k(W, idx)


def kernel(x_flat, W):
    idx = _nearest_indices(x_flat, W)
    return _gather_rows(W, idx)


# speedup vs baseline: 1.0024x; 1.0024x over previous
import functools

import jax
import jax.numpy as jnp
from jax import lax
from jax.experimental import pallas as pl
from jax.experimental.pallas import tpu as pltpu
from jax.experimental.pallas import tpu_sc as plsc

N = 8192
D = 256
K = 8192
BN = 512


CK = 8192


def _argmin_body(x_ref, w_ref, idx_ref, wb_ref, wsq_ref):
    @pl.when(pl.program_id(0) == 0)
    def _():
        w = w_ref[...]
        wb_ref[...] = w.astype(jnp.bfloat16)
        wsq_ref[...] = jnp.sum(w * w, axis=1)[None, :]

    x = x_ref[...]
    xb = (x.astype(jnp.bfloat16)) * jnp.bfloat16(-2.0)
    xsq = jnp.sum(x * x, axis=1, keepdims=True)
    s_parts = [
        lax.dot_general(
            xb, wb_ref[pl.ds(c * CK, CK), :], (((1,), (1,)), ((), ())),
            preferred_element_type=jnp.float32,
        )
        for c in range(K // CK)
    ]
    RB = 64
    NSL = K // 128
    lane = lax.broadcasted_iota(jnp.int32, (RB, 128), 1).astype(jnp.float32)
    outs = []
    for r in range(BN // RB):
        xsq_r = xsq[r * RB:(r + 1) * RB, :]
        run_v = run_c = None
        for v in range(NSL):
            c, u = divmod(v, CK // 128)
            sv = s_parts[c][r * RB:(r + 1) * RB, u * 128:(u + 1) * 128]
            wsq_v = wsq_ref[:, pl.ds(v * 128, 128)]
            dv = (xsq_r + sv) + wsq_v
            if run_v is None:
                run_v = dv
                run_c = jnp.zeros((RB, 128), jnp.float32)
            else:
                better = dv < run_v
                run_v = jnp.where(better, dv, run_v)
                run_c = jnp.where(better, jnp.float32(v), run_c)
        dmin = jnp.min(run_v, axis=1, keepdims=True)
        cand = jnp.where(run_v == dmin, run_c * 128.0 + lane, jnp.float32(K))
        outs.append(jnp.min(cand, axis=1))
    idx_ref[...] = jnp.concatenate(outs, axis=0).astype(jnp.int32)


def _nearest_indices(x_flat, W):
    return pl.pallas_call(
        _argmin_body,
        grid=(N // BN,),
        in_specs=[
            pl.BlockSpec((BN, D), lambda i: (i, 0)),
            pl.BlockSpec((K, D), lambda i: (0, 0)),
        ],
        out_specs=pl.BlockSpec((BN,), lambda i: (i,)),
        out_shape=jax.ShapeDtypeStruct((N,), jnp.int32),
        scratch_shapes=[
            pltpu.VMEM((K, D), jnp.bfloat16),
            pltpu.VMEM((1, K), jnp.float32),
        ],
        compiler_params=pltpu.CompilerParams(
            dimension_semantics=("arbitrary",),
        ),
    )(x_flat, W)


def _gather_rows(W, idx):
    info = plsc.get_sparse_core_info()
    nw = info.num_cores * info.num_subcores
    bpw = N // nw
    mesh = plsc.VectorSubcoreMesh(core_axis_name="c", subcore_axis_name="s")

    h = bpw // 2

    @functools.partial(
        pl.kernel,
        mesh=mesh,
        out_type=jax.ShapeDtypeStruct((N, D), jnp.float32),
        scratch_types=[
            pltpu.VMEM((bpw,), jnp.int32),
            pltpu.VMEM((h, D), jnp.float32),
            pltpu.VMEM((h, D), jnp.float32),
            pltpu.SemaphoreType.DMA,
            pltpu.SemaphoreType.DMA,
        ],
    )
    def k(w_hbm, idx_hbm, out_hbm, idx_v, rows0_v, rows1_v, sem_g, sem_o):
        wid = lax.axis_index("s") * info.num_cores + lax.axis_index("c")
        base = wid * bpw
        pltpu.sync_copy(idx_hbm.at[pl.ds(base, bpw)], idx_v)
        c0 = pltpu.async_copy(w_hbm.at[idx_v.at[pl.ds(0, h)]], rows0_v, sem_g)
        c0.wait()
        c1 = pltpu.async_copy(w_hbm.at[idx_v.at[pl.ds(h, h)]], rows1_v, sem_g)
        o0 = pltpu.async_copy(rows0_v, out_hbm.at[pl.ds(base, h)], sem_o)
        c1.wait()
        o1 = pltpu.async_copy(rows1_v, out_hbm.at[pl.ds(base + h, h)], sem_o)
        o0.wait()
        o1.wait()

    return k(W, idx)


def kernel(x_flat, W):
    idx = _nearest_indices(x_flat, W)
    return _gather_rows(W, idx)


# speedup vs baseline: 1.0105x; 1.0081x over previous
import functools

import jax
import jax.numpy as jnp
from jax import lax
from jax.experimental import pallas as pl
from jax.experimental.pallas import tpu as pltpu
from jax.experimental.pallas import tpu_sc as plsc

N = 8192
D = 256
K = 8192
BN = 512


CK = 8192


def _argmin_body(x_ref, w_ref, idx_ref, wb_ref, wsq_ref):
    @pl.when(pl.program_id(0) == 0)
    def _():
        w = w_ref[...]
        wb_ref[...] = w.astype(jnp.bfloat16)
        wsq_ref[...] = jnp.sum(w * w, axis=1)[None, :]

    x = x_ref[...]
    xb = (x.astype(jnp.bfloat16)) * jnp.bfloat16(-2.0)
    xsq = jnp.sum(x * x, axis=1, keepdims=True)
    s_parts = [
        lax.dot_general(
            xb, wb_ref[pl.ds(c * CK, CK), :], (((1,), (1,)), ((), ())),
            preferred_element_type=jnp.float32,
        )
        for c in range(K // CK)
    ]
    RB = 64
    NSL = K // 128
    lane = lax.broadcasted_iota(jnp.int32, (RB, 128), 1).astype(jnp.float32)
    outs = []
    for r in range(BN // RB):
        xsq_r = xsq[r * RB:(r + 1) * RB, :]

        def dval(v):
            c, u = divmod(v, CK // 128)
            sv = s_parts[c][r * RB:(r + 1) * RB, u * 128:(u + 1) * 128]
            wsq_v = wsq_ref[:, pl.ds(v * 128, 128)]
            return (xsq_r + sv) + wsq_v

        run_v = run_id = None
        for p in range(NSL // 2):
            d0, d1 = dval(2 * p), dval(2 * p + 1)
            m = jnp.minimum(d0, d1)
            par = d1 < d0
            mid = jnp.where(par, jnp.float32(2 * p + 1), jnp.float32(2 * p))
            if run_v is None:
                run_v, run_id = m, mid
            else:
                better = m < run_v
                run_v = jnp.where(better, m, run_v)
                run_id = jnp.where(better, mid, run_id)
        dmin = jnp.min(run_v, axis=1, keepdims=True)
        cand = jnp.where(run_v == dmin, run_id * 128.0 + lane, jnp.float32(K))
        outs.append(jnp.min(cand, axis=1))
    idx_ref[...] = jnp.concatenate(outs, axis=0).astype(jnp.int32)


def _nearest_indices(x_flat, W):
    return pl.pallas_call(
        _argmin_body,
        grid=(N // BN,),
        in_specs=[
            pl.BlockSpec((BN, D), lambda i: (i, 0)),
            pl.BlockSpec((K, D), lambda i: (0, 0)),
        ],
        out_specs=pl.BlockSpec((BN,), lambda i: (i,)),
        out_shape=jax.ShapeDtypeStruct((N,), jnp.int32),
        scratch_shapes=[
            pltpu.VMEM((K, D), jnp.bfloat16),
            pltpu.VMEM((1, K), jnp.float32),
        ],
        compiler_params=pltpu.CompilerParams(
            dimension_semantics=("arbitrary",),
        ),
    )(x_flat, W)


def _gather_rows(W, idx):
    info = plsc.get_sparse_core_info()
    nw = info.num_cores * info.num_subcores
    bpw = N // nw
    mesh = plsc.VectorSubcoreMesh(core_axis_name="c", subcore_axis_name="s")

    h = bpw // 2

    @functools.partial(
        pl.kernel,
        mesh=mesh,
        out_type=jax.ShapeDtypeStruct((N, D), jnp.float32),
        scratch_types=[
            pltpu.VMEM((bpw,), jnp.int32),
            pltpu.VMEM((h, D), jnp.float32),
            pltpu.VMEM((h, D), jnp.float32),
            pltpu.SemaphoreType.DMA,
            pltpu.SemaphoreType.DMA,
        ],
    )
    def k(w_hbm, idx_hbm, out_hbm, idx_v, rows0_v, rows1_v, sem_g, sem_o):
        wid = lax.axis_index("s") * info.num_cores + lax.axis_index("c")
        base = wid * bpw
        pltpu.sync_copy(idx_hbm.at[pl.ds(base, bpw)], idx_v)
        c0 = pltpu.async_copy(w_hbm.at[idx_v.at[pl.ds(0, h)]], rows0_v, sem_g)
        c0.wait()
        c1 = pltpu.async_copy(w_hbm.at[idx_v.at[pl.ds(h, h)]], rows1_v, sem_g)
        o0 = pltpu.async_copy(rows0_v, out_hbm.at[pl.ds(base, h)], sem_o)
        c1.wait()
        o1 = pltpu.async_copy(rows1_v, out_hbm.at[pl.ds(base + h, h)], sem_o)
        o0.wait()
        o1.wait()

    return k(W, idx)


def kernel(x_flat, W):
    idx = _nearest_indices(x_flat, W)
    return _gather_rows(W, idx)
